# baseline (speedup 1.0000x reference)
import functools

import jax
import jax.numpy as jnp
from jax import lax
from jax.experimental import pallas as pl
from jax.experimental.pallas import tpu as pltpu

N_TOKENS = 2048
D_IN = 1024
D_OUT = 1024
N_EXP = 8
BALANCE_W = 0.01

BLK = 256
NBLK = 24
NDISP = NBLK * BLK


def _gate_kernel(x_ref, gw_ref, sc1_ref, sc2_ref, posa_ref, posb_ref,
                 be_ref, loss_ref):
    xf = x_ref[...]
    logits = lax.dot_general(xf, gw_ref[...], (((1,), (1,)), ((), ())),
                             preferred_element_type=jnp.float32)
    idx = lax.broadcasted_iota(jnp.int32, logits.shape, 1)
    big = jnp.float32(3.4e38)
    m1 = jnp.max(logits, axis=1, keepdims=True)
    i1 = jnp.min(jnp.where(logits == m1, idx, N_EXP), axis=1, keepdims=True)
    masked = jnp.where(idx == i1, -big, logits)
    m2 = jnp.max(masked, axis=1, keepdims=True)
    i2 = jnp.min(jnp.where(masked == m2, idx, N_EXP), axis=1, keepdims=True)
    s2 = 1.0 / (1.0 + jnp.exp(m1 - m2))
    s1 = 1.0 - s2
    selA = (idx == i1).astype(jnp.float32)
    selB = (idx == i2).astype(jnp.float32)

    r = lax.broadcasted_iota(jnp.int32, (N_TOKENS, N_TOKENS), 0)
    c = lax.broadcasted_iota(jnp.int32, (N_TOKENS, N_TOKENS), 1)
    tri = (c < r).astype(jnp.float32)
    rankA = lax.dot_general(tri, selA, (((1,), (0,)), ((), ())),
                            preferred_element_type=jnp.float32)
    rankB = lax.dot_general(tri, selB, (((1,), (0,)), ((), ())),
                            preferred_element_type=jnp.float32)

    cntA = jnp.sum(selA, axis=0, keepdims=True)
    cntB = jnp.sum(selB, axis=0, keepdims=True)
    cnt = cntA + cntB
    pc = jnp.float32(BLK) * jnp.ceil(cnt / BLK)
    e8r = lax.broadcasted_iota(jnp.int32, (N_EXP, N_EXP), 0)
    e8c = lax.broadcasted_iota(jnp.int32, (N_EXP, N_EXP), 1)
    tri8 = (e8r < e8c).astype(jnp.float32)
    off = lax.dot_general(pc, tri8, (((1,), (0,)), ((), ())),
                          preferred_element_type=jnp.float32)

    posA = jnp.sum(selA * (off + rankA), axis=1, keepdims=True)
    posB = jnp.sum(selB * (off + cntA + rankB), axis=1, keepdims=True)
    posa_ref[...] = posA.astype(jnp.int32)
    posb_ref[...] = posB.astype(jnp.int32)
    sc1_ref[...] = jnp.broadcast_to(s1, (N_TOKENS, 16))
    sc2_ref[...] = jnp.broadcast_to(s2, (N_TOKENS, 16))

    offs_end = off + pc
    bi = (lax.broadcasted_iota(jnp.int32, (128, N_EXP), 0)
          .astype(jnp.float32) * jnp.float32(BLK))
    cmp = (bi >= offs_end).astype(jnp.int32)
    be = jnp.sum(cmp, axis=1, keepdims=True)
    be_ref[...] = jnp.minimum(be, N_EXP - 1)

    importance = jnp.sum(selA * s1 + selB * s2, axis=0)
    load = jnp.sum(selA * (s1 > 0) + selB * (s2 > 0), axis=0)

    def cv_sq(v):
        mean = jnp.mean(v)
        var = jnp.sum((v - mean) ** 2) / (N_EXP - 1)
        return var / (mean * mean + 1e-10)

    loss = BALANCE_W * (cv_sq(importance) + cv_sq(load))
    loss_ref[...] = jnp.reshape(loss, (1, 1))


def _gate(xf, gate_W, interpret=False):
    return pl.pallas_call(
        _gate_kernel,
        in_specs=[pl.BlockSpec((N_TOKENS, D_IN), lambda: (0, 0)),
                  pl.BlockSpec((N_EXP, D_IN), lambda: (0, 0))],
        out_specs=[pl.BlockSpec((N_TOKENS, 16), lambda: (0, 0)),
                   pl.BlockSpec((N_TOKENS, 16), lambda: (0, 0)),
                   pl.BlockSpec((N_TOKENS, 1), lambda: (0, 0)),
                   pl.BlockSpec((N_TOKENS, 1), lambda: (0, 0)),
                   pl.BlockSpec((128, 1), lambda: (0, 0)),
                   pl.BlockSpec((1, 1), lambda: (0, 0))],
        out_shape=[
            jax.ShapeDtypeStruct((N_TOKENS, 16), jnp.float32),
            jax.ShapeDtypeStruct((N_TOKENS, 16), jnp.float32),
            jax.ShapeDtypeStruct((N_TOKENS, 1), jnp.int32),
            jax.ShapeDtypeStruct((N_TOKENS, 1), jnp.int32),
            jax.ShapeDtypeStruct((128, 1), jnp.int32),
            jax.ShapeDtypeStruct((1, 1), jnp.float32),
        ],
        interpret=interpret,
    )(xf, gate_W)


def _gmm_kernel(be_ref, disp_ref, ew_ref, eb_ref, out_ref):
    out_ref[...] = lax.dot_general(
        disp_ref[...], ew_ref[0], (((1,), (1,)), ((), ())),
        preferred_element_type=jnp.float32) + eb_ref[0]


def _gmm(be, disp, expert_W, expert_b, interpret=False):
    grid_spec = pltpu.PrefetchScalarGridSpec(
        num_scalar_prefetch=1,
        grid=(NBLK,),
        in_specs=[
            pl.BlockSpec((BLK, D_IN), lambda i, be: (i, 0)),
            pl.BlockSpec((1, D_OUT, D_IN), lambda i, be: (be[i], 0, 0)),
            pl.BlockSpec((1, 1, D_OUT), lambda i, be: (be[i], 0, 0)),
        ],
        out_specs=pl.BlockSpec((BLK, D_OUT), lambda i, be: (i, 0)),
    )
    return pl.pallas_call(
        _gmm_kernel,
        grid_spec=grid_spec,
        out_shape=jax.ShapeDtypeStruct((NDISP, D_OUT), jnp.float32),
        interpret=interpret,
    )(be, disp, expert_W, expert_b.reshape(N_EXP, 1, D_OUT))


from jax.experimental.pallas import tpu_sc as plsc

_SC_NUM_CORES = 2
_SC_NUM_SUBCORES = 16
NW = _SC_NUM_CORES * _SC_NUM_SUBCORES
TPW = N_TOKENS // NW
HALF = TPW // 2

def _dispatch_body(x_hbm, posa_hbm, posb_hbm, disp_hbm, ia_v, ib_v, rows_v,
                   sema, semb):
    wid = lax.axis_index("s") * _SC_NUM_CORES + lax.axis_index("c")
    base = wid * TPW
    pltpu.sync_copy(posa_hbm.at[wid], ia_v)
    pltpu.sync_copy(posb_hbm.at[wid], ib_v)
    pltpu.sync_copy(x_hbm.at[pl.ds(base, TPW)], rows_v)
    cpa = pltpu.async_copy(rows_v, disp_hbm.at[ia_v], sema)
    cpb = pltpu.async_copy(rows_v, disp_hbm.at[ib_v], semb)
    cpa.wait()
    cpb.wait()


def _combine_body(rows_hbm, posa_hbm, posb_hbm, sca_hbm, scb_hbm, y_hbm,
                  ia_v, ib_v, sa_v, sb_v, ra_v, rb_v, yv, sema, semb):
    wid = lax.axis_index("s") * _SC_NUM_CORES + lax.axis_index("c")
    base = wid * TPW
    pltpu.sync_copy(sca_hbm.at[wid], sa_v)
    pltpu.sync_copy(scb_hbm.at[wid], sb_v)
    for c in range(2):
        pltpu.sync_copy(posa_hbm.at[wid, pl.ds(c * HALF, HALF)], ia_v)
        pltpu.sync_copy(posb_hbm.at[wid, pl.ds(c * HALF, HALF)], ib_v)
        cpa = pltpu.async_copy(rows_hbm.at[ia_v], ra_v, sema)
        cpb = pltpu.async_copy(rows_hbm.at[ib_v], rb_v, semb)
        cpa.wait()
        cpb.wait()

        def body(i, _):
            tok = c * HALF + i
            sa = sa_v[tok, :]
            sb = sb_v[tok, :]
            for v in range(D_OUT // 16):
                sl = pl.ds(v * 16, 16)
                yv[i, sl] = sa * ra_v[i, sl] + sb * rb_v[i, sl]
            return 0

        lax.fori_loop(0, HALF, body, 0)
        pltpu.sync_copy(yv, y_hbm.at[pl.ds(base + c * HALF, HALF)])


@functools.cache
def _sc_kernels():
    mesh = plsc.VectorSubcoreMesh(core_axis_name="c", subcore_axis_name="s")
    dispatch = functools.partial(
        pl.kernel, mesh=mesh,
        out_type=jax.ShapeDtypeStruct((NDISP, D_IN), jnp.float32),
        scratch_types=[
            pltpu.VMEM((TPW,), jnp.int32),
            pltpu.VMEM((TPW,), jnp.int32),
            pltpu.VMEM((TPW, D_IN), jnp.float32),
            pltpu.SemaphoreType.DMA,
            pltpu.SemaphoreType.DMA,
        ],
    )(_dispatch_body)
    combine = functools.partial(
        pl.kernel, mesh=mesh,
        out_type=jax.ShapeDtypeStruct((N_TOKENS, D_OUT), jnp.float32),
        scratch_types=[
            pltpu.VMEM((HALF,), jnp.int32),
            pltpu.VMEM((HALF,), jnp.int32),
            pltpu.VMEM((TPW, 16), jnp.float32),
            pltpu.VMEM((TPW, 16), jnp.float32),
            pltpu.VMEM((HALF, D_OUT), jnp.float32),
            pltpu.VMEM((HALF, D_OUT), jnp.float32),
            pltpu.VMEM((HALF, D_OUT), jnp.float32),
            pltpu.SemaphoreType.DMA,
            pltpu.SemaphoreType.DMA,
        ],
    )(_combine_body)
    return dispatch, combine


def _route_sim(xf, sc1, sc2, posA, posB):
    disp = jnp.zeros((NDISP, D_IN), jnp.float32)
    disp = disp.at[posA].set(xf)
    disp = disp.at[posB].set(xf)
    return disp


def _combine_sim(rows, sc1, sc2, posA, posB):
    return sc1[:, None] * rows[posA] + sc2[:, None] * rows[posB]


@jax.jit
def _moe_routed(xf, gate_W, expert_W, expert_b):
    sc1, sc2, posA, posB, be, loss = _gate(xf, gate_W)
    posa32 = posA.reshape(NW, TPW)
    posb32 = posB.reshape(NW, TPW)
    dispatch, combine = _sc_kernels()
    disp = dispatch(xf, posa32, posb32)
    rows = _gmm(be[:NBLK, 0], disp, expert_W, expert_b)
    y = combine(rows, posa32, posb32,
                sc1.reshape(NW, TPW, 16), sc2.reshape(NW, TPW, 16))
    return y, loss


def kernel(x, gate_W, expert_W, expert_b):
    orig_shape = x.shape[:-1]
    xf = x.reshape(-1, D_IN)
    y, loss = _moe_routed(xf, gate_W, expert_W, expert_b)
    return y.reshape(orig_shape + (D_OUT,)), loss[0, 0]


def kernel_cpu_test(x, gate_W, expert_W, expert_b):
    xf = x.reshape(-1, D_IN)
    sc1, sc2, posA, posB, be, loss = _gate(xf, gate_W, interpret=True)
    sc1, sc2 = sc1[:, 0], sc2[:, 0]
    posA, posB, be = posA[:, 0], posB[:, 0], be[:24, 0]
    disp = _route_sim(xf, sc1, sc2, posA, posB)
    rows = _gmm(be, disp, expert_W, expert_b, interpret=True)
    y = _combine_sim(rows, sc1, sc2, posA, posB)
    return y.reshape(x.shape[:-1] + (D_OUT,)), loss[0, 0]


# speedup vs baseline: 1.8548x; 1.8548x over previous
import functools

import jax
import jax.numpy as jnp
from jax import lax
from jax.experimental import pallas as pl
from jax.experimental.pallas import tpu as pltpu

N_TOKENS = 2048
D_IN = 1024
D_OUT = 1024
N_EXP = 8
BALANCE_W = 0.01

TBLK = 256
NT = N_TOKENS // TBLK


def _moe_kernel(x_ref, gw_ref, ew_ref, eb_ref, y_ref, loss_ref,
                imp_ref, load_ref):
    t = pl.program_id(0)
    xb = x_ref[...]

    logits = lax.dot_general(xb, gw_ref[...], (((1,), (1,)), ((), ())),
                             preferred_element_type=jnp.float32)
    idx = lax.broadcasted_iota(jnp.int32, logits.shape, 1)
    big = jnp.float32(3.4e38)
    m1 = jnp.max(logits, axis=1, keepdims=True)
    i1 = jnp.min(jnp.where(logits == m1, idx, N_EXP), axis=1, keepdims=True)
    masked = jnp.where(idx == i1, -big, logits)
    m2 = jnp.max(masked, axis=1, keepdims=True)
    i2 = jnp.min(jnp.where(masked == m2, idx, N_EXP), axis=1, keepdims=True)
    s2 = 1.0 / (1.0 + jnp.exp(m1 - m2))
    s1 = 1.0 - s2
    sc = jnp.where(idx == i1, s1, jnp.where(idx == i2, s2, 0.0))

    ew = ew_ref[...].reshape(N_EXP * D_OUT, D_IN)
    xw = lax.dot_general(xb, ew, (((1,), (1,)), ((), ())),
                         preferred_element_type=jnp.float32)
    y = lax.dot_general(sc, eb_ref[...], (((1,), (0,)), ((), ())),
                        preferred_element_type=jnp.float32)
    for e in range(N_EXP):
        y = y + sc[:, e][:, None] * xw[:, e * D_OUT:(e + 1) * D_OUT]
    y_ref[...] = y

    imp_blk = jnp.sum(sc, axis=0, keepdims=True)
    load_blk = jnp.sum((sc > 0).astype(jnp.float32), axis=0, keepdims=True)

    @pl.when(t == 0)
    def _init():
        imp_ref[...] = imp_blk
        load_ref[...] = load_blk

    @pl.when(t > 0)
    def _acc():
        imp_ref[...] += imp_blk
        load_ref[...] += load_blk

    def cv_sq(v):
        mean = jnp.mean(v)
        var = jnp.sum((v - mean) ** 2) / (N_EXP - 1)
        return var / (mean * mean + 1e-10)

    loss = BALANCE_W * (cv_sq(imp_ref[0, :]) + cv_sq(load_ref[0, :]))
    loss_ref[...] = jnp.reshape(loss, (1, 1))


@functools.partial(jax.jit)
def _moe(xf, gate_W, expert_W, expert_b):
    y, loss = pl.pallas_call(
        _moe_kernel,
        grid=(NT,),
        in_specs=[
            pl.BlockSpec((TBLK, D_IN), lambda t: (t, 0)),
            pl.BlockSpec((N_EXP, D_IN), lambda t: (0, 0)),
            pl.BlockSpec((N_EXP, D_OUT, D_IN), lambda t: (0, 0, 0)),
            pl.BlockSpec((N_EXP, D_OUT), lambda t: (0, 0)),
        ],
        out_specs=[
            pl.BlockSpec((TBLK, D_OUT), lambda t: (t, 0)),
            pl.BlockSpec((1, 1), lambda t: (0, 0)),
        ],
        out_shape=[
            jax.ShapeDtypeStruct((N_TOKENS, D_OUT), jnp.float32),
            jax.ShapeDtypeStruct((1, 1), jnp.float32),
        ],
        scratch_shapes=[
            pltpu.VMEM((1, N_EXP), jnp.float32),
            pltpu.VMEM((1, N_EXP), jnp.float32),
        ],
    )(xf, gate_W, expert_W, expert_b)
    return y, loss


def kernel(x, gate_W, expert_W, expert_b):
    orig_shape = x.shape[:-1]
    xf = x.reshape(-1, D_IN)
    y, loss = _moe(xf, gate_W, expert_W, expert_b)
    return y.reshape(orig_shape + (D_OUT,)), loss[0, 0]


# speedup vs baseline: 1.9448x; 1.0485x over previous
import functools

import jax
import jax.numpy as jnp
from jax import lax
from jax.experimental import pallas as pl
from jax.experimental.pallas import tpu as pltpu

N_TOKENS = 2048
D_IN = 1024
D_OUT = 1024
N_EXP = 8
BALANCE_W = 0.01
HALF_O = D_OUT // 2


def _moe_kernel(x_ref, gw_ref, ewa_ref, ewb_ref, eb_ref, y_ref, loss_ref,
                scores_ref):
    e = pl.program_id(0)

    @pl.when(e == 0)
    def _init():
        xf = x_ref[...]
        logits = lax.dot_general(
            xf, gw_ref[...], (((1,), (1,)), ((), ())),
            preferred_element_type=jnp.float32)
        idx = lax.broadcasted_iota(jnp.int32, logits.shape, 1)
        big = jnp.float32(3.4e38)
        m1 = jnp.max(logits, axis=1, keepdims=True)
        i1 = jnp.min(jnp.where(logits == m1, idx, N_EXP), axis=1,
                     keepdims=True)
        masked = jnp.where(idx == i1, -big, logits)
        m2 = jnp.max(masked, axis=1, keepdims=True)
        i2 = jnp.min(jnp.where(masked == m2, idx, N_EXP), axis=1,
                     keepdims=True)
        s2 = 1.0 / (1.0 + jnp.exp(m1 - m2))
        s1 = 1.0 - s2
        scores_ref[...] = jnp.where(
            idx == i1, s1, jnp.where(idx == i2, s2, 0.0))

    xf = x_ref[...]
    xwa = lax.dot_general(xf, ewa_ref[0], (((1,), (1,)), ((), ())),
                          preferred_element_type=jnp.float32)
    xwb = lax.dot_general(xf, ewb_ref[0], (((1,), (1,)), ((), ())),
                          preferred_element_type=jnp.float32)
    sc = scores_ref[...]

    for k in range(N_EXP):
        @pl.when(e == k)
        def _apply(k=k):
            s_col = sc[:, k:k + 1]
            if k == 0:
                y_ref[:, :HALF_O] = s_col * xwa
                y_ref[:, HALF_O:] = s_col * xwb
            else:
                y_ref[:, :HALF_O] += s_col * xwa
                y_ref[:, HALF_O:] += s_col * xwb

    @pl.when(e == N_EXP - 1)
    def _fini():
        y_ref[...] += lax.dot_general(
            sc, eb_ref[...], (((1,), (0,)), ((), ())),
            preferred_element_type=jnp.float32)

        importance = jnp.sum(sc, axis=0)
        load = jnp.sum((sc > 0).astype(jnp.float32), axis=0)

        def cv_sq(v):
            mean = jnp.mean(v)
            var = jnp.sum((v - mean) ** 2) / (N_EXP - 1)
            return var / (mean * mean + 1e-10)

        loss = BALANCE_W * (cv_sq(importance) + cv_sq(load))
        loss_ref[...] = jnp.reshape(loss, (1, 1))


@functools.partial(jax.jit)
def _moe(xf, gate_W, expert_W, expert_b):
    y, loss = pl.pallas_call(
        _moe_kernel,
        grid=(N_EXP,),
        in_specs=[
            pl.BlockSpec((N_TOKENS, D_IN), lambda e: (0, 0)),
            pl.BlockSpec((N_EXP, D_IN), lambda e: (0, 0)),
            pl.BlockSpec((1, HALF_O, D_IN), lambda e: (e, 0, 0)),
            pl.BlockSpec((1, HALF_O, D_IN), lambda e: (e, 1, 0)),
            pl.BlockSpec((N_EXP, D_OUT), lambda e: (0, 0)),
        ],
        out_specs=[
            pl.BlockSpec((N_TOKENS, D_OUT), lambda e: (0, 0)),
            pl.BlockSpec((1, 1), lambda e: (0, 0)),
        ],
        out_shape=[
            jax.ShapeDtypeStruct((N_TOKENS, D_OUT), jnp.float32),
            jax.ShapeDtypeStruct((1, 1), jnp.float32),
        ],
        scratch_shapes=[pltpu.VMEM((N_TOKENS, N_EXP), jnp.float32)],
    )(xf, gate_W, expert_W, expert_W, expert_b)
    return y, loss


def kernel(x, gate_W, expert_W, expert_b):
    orig_shape = x.shape[:-1]
    xf = x.reshape(-1, D_IN)
    y, loss = _moe(xf, gate_W, expert_W, expert_b)
    return y.reshape(orig_shape + (D_OUT,)), loss[0, 0]
